# baseline (speedup 1.0000x reference)
import functools

import jax
import jax.numpy as jnp
from jax import lax
from jax.experimental import pallas as pl
from jax.experimental.pallas import tpu as pltpu
from jax.experimental.pallas import tpu_sc as plsc

NUM_ROWS = 256
NUM_COLS = 32
EPS = 1e-10
LOG_ZERO = -10000.0

B = 4
S = 8192
H = 768
NCELL = NUM_ROWS * NUM_COLS
L = 16


_TL_BLK = 4096
_TL_GRID = (B * S) // _TL_BLK


def _tl_body(x_ref, w_ref, b_ref, o_ref):
    x = x_ref[0]
    w2 = w_ref[...].reshape(1, H)
    y = jax.lax.dot_general(
        w2, x, (((1,), (1,)), ((), ())),
        preferred_element_type=jnp.float32)
    o_ref[...] = y[None] + b_ref[0]


def _token_logits_half(x3, w, b, base):
    half = _TL_GRID // 2
    out = pl.pallas_call(
        _tl_body,
        grid=(half,),
        in_specs=[
            pl.BlockSpec((1, _TL_BLK, H), lambda i: (i + base, 0, 0)),
            pl.BlockSpec((H,), lambda i: (0,)),
            pl.BlockSpec(memory_space=pltpu.SMEM),
        ],
        out_specs=pl.BlockSpec((1, 1, _TL_BLK), lambda i: (i, 0, 0)),
        out_shape=jax.ShapeDtypeStruct((half, 1, _TL_BLK), jnp.float32),
    )(x3, w, b.reshape(1))
    return out.reshape(2, S)


_sc_mesh = plsc.VectorSubcoreMesh(
    core_axis_name="c", subcore_axis_name="s", num_cores=2, num_subcores=16
)

_G = 16
_TOK = S // _G
_CEL = NCELL // _G


def _make_sc_pair(off):
    @functools.partial(
        pl.kernel,
        out_type=jax.ShapeDtypeStruct((2, NUM_COLS), jnp.float32),
        mesh=_sc_mesh,
        compiler_params=pltpu.CompilerParams(needs_layout_passes=False),
        scratch_types=[
            pltpu.VMEM((_TOK,), jnp.float32),
            pltpu.VMEM((_TOK,), jnp.int32),
            pltpu.VMEM((_TOK,), jnp.int32),
            pltpu.VMEM((_CEL,), jnp.float32),
            pltpu.VMEM((NCELL,), jnp.float32),
            pltpu.VMEM((NCELL,), jnp.float32),
            pltpu.VMEM((_G, _CEL), jnp.float32),
            pltpu.VMEM((_G, _CEL), jnp.float32),
            pltpu.VMEM((_G, 128), jnp.float32),
            pltpu.VMEM((128,), jnp.float32),
            pltpu.VMEM((NUM_COLS,), jnp.float32),
            pltpu.VMEM((NUM_COLS,), jnp.float32),
            pltpu.VMEM_SHARED((_G, 2, NCELL), jnp.float32),
            pltpu.VMEM_SHARED((_G, 128), jnp.float32),
            pltpu.SemaphoreType.DMA,
        ],
    )
    def _sc_pair(tl_hbm, row_hbm, col_hbm, mask_hbm, pen_hbm, out_hbm,
                 tl_v, row_v, col_v, mask_v, sums_v, cnts_v,
                 psum_v, pcnt_v, pcol_v, stage_v, pen_v, out_v,
                 bins_sh, cols_sh, dma_sem):
        cid = lax.axis_index("c")
        g = lax.axis_index("s")
        b = off + cid
        bl = cid

        cps = [
            pltpu.async_copy(tl_hbm.at[bl, pl.ds(g * _TOK, _TOK)], tl_v,
                             dma_sem),
            pltpu.async_copy(row_hbm.at[b, pl.ds(g * _TOK, _TOK)], row_v,
                             dma_sem),
            pltpu.async_copy(col_hbm.at[b, pl.ds(g * _TOK, _TOK)], col_v,
                             dma_sem),
            pltpu.async_copy(mask_hbm.at[b, pl.ds(g * _CEL, _CEL)], mask_v,
                             dma_sem),
            pltpu.async_copy(pen_hbm, pen_v, dma_sem),
        ]
        for cp in cps:
            cp.wait()

        zeros = jnp.zeros((L,), jnp.float32)
        ones = jnp.full((L,), 1.0, jnp.float32)

        def zero_body(i, _):
            sums_v[pl.ds(i * L, L)] = zeros
            cnts_v[pl.ds(i * L, L)] = zeros
            return 0

        lax.fori_loop(0, NCELL // L, zero_body, 0, unroll=8)

        def scat_body(i, _):
            r = row_v[pl.ds(i * L, L)]
            c = col_v[pl.ds(i * L, L)]
            v = tl_v[pl.ds(i * L, L)]
            idx = c + r * NUM_COLS
            plsc.addupdate_scatter(sums_v, [idx], v)
            plsc.addupdate_scatter(cnts_v, [idx], ones)
            return 0

        lax.fori_loop(0, _TOK // L, scat_body, 0, unroll=8)

        cps = [
            pltpu.async_copy(sums_v, bins_sh.at[g, 0], dma_sem),
            pltpu.async_copy(cnts_v, bins_sh.at[g, 1], dma_sem),
        ]
        for cp in cps:
            cp.wait()
        plsc.subcore_barrier()
        cps = []
        for t in range(_G):
            cps.append(pltpu.async_copy(
                bins_sh.at[t, 0, pl.ds(g * _CEL, _CEL)], psum_v.at[t],
                dma_sem))
            cps.append(pltpu.async_copy(
                bins_sh.at[t, 1, pl.ds(g * _CEL, _CEL)], pcnt_v.at[t],
                dma_sem))
        for cp in cps:
            cp.wait()

        def red_body(i, carry):
            a0, a1, m0, m1 = carry
            base = i * 2 * L
            s0 = psum_v[0, pl.ds(base, L)]
            c0 = pcnt_v[0, pl.ds(base, L)]
            s1 = psum_v[0, pl.ds(base + L, L)]
            c1 = pcnt_v[0, pl.ds(base + L, L)]
            for t in range(1, _G):
                s0 = s0 + psum_v[t, pl.ds(base, L)]
                c0 = c0 + pcnt_v[t, pl.ds(base, L)]
                s1 = s1 + psum_v[t, pl.ds(base + L, L)]
                c1 = c1 + pcnt_v[t, pl.ds(base + L, L)]
            k0 = mask_v[pl.ds(base, L)]
            k1 = mask_v[pl.ds(base + L, L)]
            a0 = a0 + k0 * (s0 / jnp.maximum(c0, 1.0))
            m0 = m0 + k0
            a1 = a1 + k1 * (s1 / jnp.maximum(c1, 1.0))
            m1 = m1 + k1
            return (a0, a1, m0, m1)

        a0, a1, m0, m1 = lax.fori_loop(
            0, _CEL // (2 * L), red_body,
            (zeros, zeros, zeros, zeros), unroll=4)

        stage_v[pl.ds(0, L)] = a0
        stage_v[pl.ds(L, L)] = a1
        stage_v[pl.ds(2 * L, L)] = m0
        stage_v[pl.ds(3 * L, L)] = m1
        for i in range(4, 8):
            stage_v[pl.ds(i * L, L)] = zeros
        pltpu.sync_copy(stage_v, cols_sh.at[g])
        plsc.subcore_barrier()

        @pl.when(g == 0)
        def _():
            pltpu.sync_copy(cols_sh, pcol_v)
            f0 = pcol_v[0, pl.ds(0, L)]
            f1 = pcol_v[0, pl.ds(L, L)]
            n0 = pcol_v[0, pl.ds(2 * L, L)]
            n1 = pcol_v[0, pl.ds(3 * L, L)]
            for t in range(1, _G):
                f0 = f0 + pcol_v[t, pl.ds(0, L)]
                f1 = f1 + pcol_v[t, pl.ds(L, L)]
                n0 = n0 + pcol_v[t, pl.ds(2 * L, L)]
                n1 = n1 + pcol_v[t, pl.ds(3 * L, L)]
            lane = lax.broadcasted_iota(jnp.int32, (L,), 0)
            pad0 = jnp.logical_and(n0 < 0.5, lane != 0).astype(jnp.float32)
            pad1 = (n1 < 0.5).astype(jnp.float32)
            out0 = f0 / (n0 + EPS) + LOG_ZERO * pad0 + pen_v[pl.ds(0, L)]
            out1 = f1 / (n1 + EPS) + LOG_ZERO * pad1 + pen_v[pl.ds(L, L)]
            out_v[pl.ds(0, L)] = out0
            out_v[pl.ds(L, L)] = out1
            pltpu.sync_copy(out_v, out_hbm.at[bl])

    return _sc_pair


_sc_pair_lo = _make_sc_pair(0)
_sc_pair_hi = _make_sc_pair(2)


def kernel(sequence_output, row_ids, col_ids, cell_mask, column_output_weights,
           column_output_bias, allow_empty_column_selection):
    x3 = sequence_output.reshape(_TL_GRID, _TL_BLK, H)
    w = column_output_weights
    bias = column_output_bias.astype(jnp.float32)
    rows = row_ids.astype(jnp.int32)
    cols = col_ids.astype(jnp.int32)
    col0 = (jnp.arange(NUM_COLS) == 0).astype(jnp.float32)
    pen = (jnp.where(allow_empty_column_selection == 0, LOG_ZERO, 0.0)
           * col0).astype(jnp.float32)

    tl_lo = _token_logits_half(x3, w, bias, 0)
    out_lo = _sc_pair_lo(tl_lo, rows, cols, cell_mask, pen)
    tl_hi = _token_logits_half(x3, w, bias, _TL_GRID // 2)
    out_hi = _sc_pair_hi(tl_hi, rows, cols, cell_mask, pen)
    return jnp.concatenate([out_lo, out_hi], axis=0)


# speedup vs baseline: 1.1294x; 1.1294x over previous
import functools

import jax
import jax.numpy as jnp
from jax import lax
from jax.experimental import pallas as pl
from jax.experimental.pallas import tpu as pltpu
from jax.experimental.pallas import tpu_sc as plsc

NUM_ROWS = 256
NUM_COLS = 32
EPS = 1e-10
LOG_ZERO = -10000.0

B = 4
S = 8192
H = 768
NCELL = NUM_ROWS * NUM_COLS
L = 16


_TL_BLK = 4096
_TL_GRID = (B * S) // _TL_BLK


def _tl_body(x_ref, w_ref, b_ref, o_ref):
    x = x_ref[0]
    w2 = w_ref[...].reshape(1, H)
    y = jax.lax.dot_general(
        w2, x, (((1,), (1,)), ((), ())),
        preferred_element_type=jnp.float32)
    o_ref[...] = y[None] + b_ref[0]


def _token_logits(seq_flat, w, b):
    x3 = seq_flat.reshape(_TL_GRID, _TL_BLK, H)
    out = pl.pallas_call(
        _tl_body,
        grid=(_TL_GRID,),
        in_specs=[
            pl.BlockSpec((1, _TL_BLK, H), lambda i: (i, 0, 0)),
            pl.BlockSpec((H,), lambda i: (0,)),
            pl.BlockSpec(memory_space=pltpu.SMEM),
        ],
        out_specs=pl.BlockSpec((1, 1, _TL_BLK), lambda i: (i, 0, 0)),
        out_shape=jax.ShapeDtypeStruct((_TL_GRID, 1, _TL_BLK), jnp.float32),
    )(x3, w, b.reshape(1))
    return out.reshape(B, S)


_sc_mesh = plsc.VectorSubcoreMesh(
    core_axis_name="c", subcore_axis_name="s", num_cores=2, num_subcores=16
)

_G = 8
_TOK = S // _G
_CEL = NCELL // _G


@functools.partial(
    pl.kernel,
    out_type=jax.ShapeDtypeStruct((B, NUM_COLS), jnp.float32),
    mesh=_sc_mesh,
    compiler_params=pltpu.CompilerParams(needs_layout_passes=False),
    scratch_types=[
        pltpu.VMEM((_TOK,), jnp.float32),
        pltpu.VMEM((_TOK,), jnp.int32),
        pltpu.VMEM((_TOK,), jnp.int32),
        pltpu.VMEM((_CEL,), jnp.float32),
        pltpu.VMEM((NCELL,), jnp.float32),
        pltpu.VMEM((NCELL,), jnp.float32),
        pltpu.VMEM((_G, _CEL), jnp.float32),
        pltpu.VMEM((_G, _CEL), jnp.float32),
        pltpu.VMEM((_G, 128), jnp.float32),
        pltpu.VMEM((128,), jnp.float32),
        pltpu.VMEM((NUM_COLS,), jnp.float32),
        pltpu.VMEM((NUM_COLS,), jnp.float32),
        pltpu.VMEM_SHARED((2, _G, 2, NCELL), jnp.float32),
        pltpu.VMEM_SHARED((2, _G, 128), jnp.float32),
        pltpu.SemaphoreType.DMA,
    ],
)
def _sc_columns(tl_hbm, row_hbm, col_hbm, mask_hbm, pen_hbm, out_hbm,
                tl_v, row_v, col_v, mask_v, sums_v, cnts_v,
                psum_v, pcnt_v, pcol_v, stage_v, pen_v, out_v, bins_sh, cols_sh,
                dma_sem):
    cid = lax.axis_index("c")
    sid = lax.axis_index("s")
    bs = sid // _G
    g = sid % _G
    b = cid * 2 + bs

    cps = [
        pltpu.async_copy(tl_hbm.at[b, pl.ds(g * _TOK, _TOK)], tl_v, dma_sem),
        pltpu.async_copy(row_hbm.at[b, pl.ds(g * _TOK, _TOK)], row_v, dma_sem),
        pltpu.async_copy(col_hbm.at[b, pl.ds(g * _TOK, _TOK)], col_v, dma_sem),
        pltpu.async_copy(mask_hbm.at[b, pl.ds(g * _CEL, _CEL)], mask_v, dma_sem),
        pltpu.async_copy(pen_hbm, pen_v, dma_sem),
    ]
    for cp in cps:
        cp.wait()

    zeros = jnp.zeros((L,), jnp.float32)
    ones = jnp.full((L,), 1.0, jnp.float32)

    def zero_body(i, _):
        sums_v[pl.ds(i * L, L)] = zeros
        cnts_v[pl.ds(i * L, L)] = zeros
        return 0

    lax.fori_loop(0, NCELL // L, zero_body, 0, unroll=8)

    def scat_body(i, _):
        r = row_v[pl.ds(i * L, L)]
        c = col_v[pl.ds(i * L, L)]
        v = tl_v[pl.ds(i * L, L)]
        idx = c + r * NUM_COLS
        plsc.addupdate_scatter(sums_v, [idx], v)
        plsc.addupdate_scatter(cnts_v, [idx], ones)
        return 0

    lax.fori_loop(0, _TOK // L, scat_body, 0, unroll=8)

    cps = [
        pltpu.async_copy(sums_v, bins_sh.at[bs, g, 0], dma_sem),
        pltpu.async_copy(cnts_v, bins_sh.at[bs, g, 1], dma_sem),
    ]
    for cp in cps:
        cp.wait()
    plsc.subcore_barrier()
    cps = []
    for t in range(_G):
        cps.append(pltpu.async_copy(
            bins_sh.at[bs, t, 0, pl.ds(g * _CEL, _CEL)], psum_v.at[t], dma_sem))
        cps.append(pltpu.async_copy(
            bins_sh.at[bs, t, 1, pl.ds(g * _CEL, _CEL)], pcnt_v.at[t], dma_sem))
    for cp in cps:
        cp.wait()

    def red_body(i, carry):
        a0, a1, m0, m1 = carry
        base = i * 2 * L
        s0 = psum_v[0, pl.ds(base, L)]
        c0 = pcnt_v[0, pl.ds(base, L)]
        s1 = psum_v[0, pl.ds(base + L, L)]
        c1 = pcnt_v[0, pl.ds(base + L, L)]
        for t in range(1, _G):
            s0 = s0 + psum_v[t, pl.ds(base, L)]
            c0 = c0 + pcnt_v[t, pl.ds(base, L)]
            s1 = s1 + psum_v[t, pl.ds(base + L, L)]
            c1 = c1 + pcnt_v[t, pl.ds(base + L, L)]
        k0 = mask_v[pl.ds(base, L)]
        k1 = mask_v[pl.ds(base + L, L)]
        a0 = a0 + k0 * (s0 / jnp.maximum(c0, 1.0))
        m0 = m0 + k0
        a1 = a1 + k1 * (s1 / jnp.maximum(c1, 1.0))
        m1 = m1 + k1
        return (a0, a1, m0, m1)

    a0, a1, m0, m1 = lax.fori_loop(
        0, _CEL // (2 * L), red_body,
        (zeros, zeros, zeros, zeros), unroll=4)

    stage_v[pl.ds(0, L)] = a0
    stage_v[pl.ds(L, L)] = a1
    stage_v[pl.ds(2 * L, L)] = m0
    stage_v[pl.ds(3 * L, L)] = m1
    for i in range(4, 8):
        stage_v[pl.ds(i * L, L)] = zeros
    pltpu.sync_copy(stage_v, cols_sh.at[bs, g])
    plsc.subcore_barrier()

    @pl.when(g == 0)
    def _():
        pltpu.sync_copy(cols_sh.at[bs], pcol_v)
        f0 = pcol_v[0, pl.ds(0, L)]
        f1 = pcol_v[0, pl.ds(L, L)]
        n0 = pcol_v[0, pl.ds(2 * L, L)]
        n1 = pcol_v[0, pl.ds(3 * L, L)]
        for t in range(1, _G):
            f0 = f0 + pcol_v[t, pl.ds(0, L)]
            f1 = f1 + pcol_v[t, pl.ds(L, L)]
            n0 = n0 + pcol_v[t, pl.ds(2 * L, L)]
            n1 = n1 + pcol_v[t, pl.ds(3 * L, L)]
        lane = lax.broadcasted_iota(jnp.int32, (L,), 0)
        pad0 = jnp.logical_and(n0 < 0.5, lane != 0).astype(jnp.float32)
        pad1 = (n1 < 0.5).astype(jnp.float32)
        out0 = f0 / (n0 + EPS) + LOG_ZERO * pad0 + pen_v[pl.ds(0, L)]
        out1 = f1 / (n1 + EPS) + LOG_ZERO * pad1 + pen_v[pl.ds(L, L)]
        out_v[pl.ds(0, L)] = out0
        out_v[pl.ds(L, L)] = out1
        pltpu.sync_copy(out_v, out_hbm.at[b])


def kernel(sequence_output, row_ids, col_ids, cell_mask, column_output_weights,
           column_output_bias, allow_empty_column_selection):
    tl = _token_logits(
        sequence_output.reshape(B * S, H),
        column_output_weights,
        column_output_bias.astype(jnp.float32),
    )
    col0 = (jnp.arange(NUM_COLS) == 0).astype(jnp.float32)
    pen = jnp.where(allow_empty_column_selection == 0, LOG_ZERO, 0.0) * col0
    return _sc_columns(tl, row_ids.astype(jnp.int32), col_ids.astype(jnp.int32),
                       cell_mask, pen.astype(jnp.float32))
